# baseline (speedup 1.0000x reference)
import functools

import jax
import jax.numpy as jnp
from jax import lax
from jax.experimental import pallas as pl
from jax.experimental.pallas import tpu as pltpu
from jax.experimental.pallas import tpu_sc as plsc

_LANES = 16
_SUBL = 8


def _build_consts(thresholds):
    F, T = thresholds.shape
    d = jnp.diff(thresholds, axis=1)
    d = jnp.concatenate([-d[:, :1], d, d[:, -1:]], axis=1)
    th_full = jnp.concatenate([thresholds[:, :1], thresholds], axis=1)
    tr = th_full + d / 2.0
    pad = jnp.full((F, _LANES - T), jnp.inf, jnp.float32)
    return jnp.concatenate([thresholds, pad, tr], axis=1)


@functools.partial(jax.jit, static_argnames=("cols_per_w", "chunk"))
def _run(xt, consts, cols_per_w, chunk):
    F, B = xt.shape
    n_chunks = cols_per_w // chunk
    mesh = plsc.VectorSubcoreMesh(core_axis_name="c", subcore_axis_name="s")

    @functools.partial(
        pl.kernel,
        mesh=mesh,
        out_type=jax.ShapeDtypeStruct((F, B), jnp.float32),
        compiler_params=pltpu.CompilerParams(needs_layout_passes=False),
        scratch_types=(
            [pltpu.VMEM((_LANES,), jnp.float32) for _ in range(2 * _SUBL)]
            + [pltpu.VMEM((_SUBL, chunk), jnp.float32) for _ in range(4)]
            + [pltpu.SemaphoreType.DMA for _ in range(4)]
        ),
    )
    def run(x_hbm, c_hbm, out_hbm, *scratch):
        th_refs = scratch[:_SUBL]
        tr_refs = scratch[_SUBL:2 * _SUBL]
        xb0, xb1, ob0, ob1 = scratch[2 * _SUBL:2 * _SUBL + 4]
        si0, si1, so0, so1 = scratch[2 * _SUBL + 4:]

        wid = lax.axis_index("s") * 2 + lax.axis_index("c")
        fr = wid % 2
        f0 = fr * _SUBL
        col0 = (wid // 2) * cols_per_w
        for s in range(_SUBL):
            pltpu.sync_copy(c_hbm.at[f0 + s, pl.ds(0, _LANES)], th_refs[s])
            pltpu.sync_copy(c_hbm.at[f0 + s, pl.ds(_LANES, _LANES)],
                            tr_refs[s])

        def compute(xbuf, obuf):
            for s in range(_SUBL):
                th_v, tr_v = th_refs[s], tr_refs[s]
                thv = th_v[...]
                th7, th3, th11 = thv[7], thv[3], thv[11]

                @plsc.parallel_loop(0, chunk // _LANES, 1, unroll=4)
                def vec_body(r):
                    xv = xbuf[s, pl.ds(r * _LANES, _LANES)]
                    m0 = xv > th7
                    enc = jnp.where(m0, jnp.int32(8), jnp.int32(0))
                    pv = jnp.where(m0, th11, th3)
                    enc = jnp.where(xv > pv, enc + 4, enc)
                    pv = plsc.load_gather(th_v, [enc + 1])
                    enc = jnp.where(xv > pv, enc + 2, enc)
                    pv = plsc.load_gather(th_v, [enc])
                    enc = jnp.where(xv > pv, enc + 1, enc)
                    obuf[s, pl.ds(r * _LANES, _LANES)] = (
                        plsc.load_gather(tr_v, [enc]))

        def drain(sem, buf):
            pltpu.make_async_copy(
                x_hbm.at[pl.ds(f0, _SUBL), pl.ds(col0, chunk)], buf,
                sem).wait()

        last = col0 + (n_chunks - 1) * chunk

        def in_copy(c, buf, sem):
            return pltpu.async_copy(
                x_hbm.at[pl.ds(f0, _SUBL), pl.ds(c, chunk)], buf, sem)

        def out_copy(buf, c, sem):
            return pltpu.async_copy(
                buf, out_hbm.at[pl.ds(f0, _SUBL), pl.ds(c, chunk)], sem)

        in_copy(col0, xb0, si0)
        in_copy(col0 + chunk, xb1, si1)

        def pair_body(g, carry):
            a0 = col0 + (2 * g) * chunk

            drain(si0, xb0)

            @pl.when(g > 0)
            def _():
                drain(so0, ob0)

            compute(xb0, ob0)
            out_copy(ob0, a0, so0)
            in_copy(jnp.minimum(a0 + 2 * chunk, last), xb0, si0)

            drain(si1, xb1)

            @pl.when(g > 0)
            def _():
                drain(so1, ob1)

            compute(xb1, ob1)
            out_copy(ob1, a0 + chunk, so1)
            in_copy(jnp.minimum(a0 + 3 * chunk, last), xb1, si1)
            return carry

        lax.fori_loop(0, n_chunks // 2, pair_body, 0)
        drain(si0, xb0)
        drain(si1, xb1)
        drain(so0, ob0)
        drain(so1, ob1)

    return run(xt, consts)


def kernel(x, thresholds):
    B, F = x.shape
    consts = _build_consts(thresholds)
    info = plsc.get_sparse_core_info()
    n_workers = info.num_cores * info.num_subcores
    cols_per_w = B // (n_workers // 2)
    chunk = 2048
    out_t = _run(x.T, consts, cols_per_w, chunk)
    return out_t.T


# speedup vs baseline: 1.2971x; 1.2971x over previous
import functools

import jax
import jax.numpy as jnp
from jax import lax
from jax.experimental import pallas as pl
from jax.experimental.pallas import tpu as pltpu
from jax.experimental.pallas import tpu_sc as plsc

_LANES = 16


def _build_consts(thresholds):
    F, T = thresholds.shape
    d = jnp.diff(thresholds, axis=1)
    d = jnp.concatenate([-d[:, :1], d, d[:, -1:]], axis=1)
    th_full = jnp.concatenate([thresholds[:, :1], thresholds], axis=1)
    tr = th_full + d / 2.0
    pad = jnp.full((F, _LANES - T), jnp.inf, jnp.float32)
    pad1 = jnp.full((F, _LANES - T + 1), jnp.inf, jnp.float32)
    th_shift = jnp.concatenate([thresholds[:, 1:], pad1], axis=1)
    return jnp.concatenate([thresholds, pad, th_shift, tr], axis=1)


@functools.partial(jax.jit, static_argnames=("cols_per_w", "chunk"))
def _run(xt, consts, cols_per_w, chunk):
    F, B = xt.shape
    n_chunks = cols_per_w // chunk
    w_per_f = B // cols_per_w
    mesh = plsc.VectorSubcoreMesh(core_axis_name="c", subcore_axis_name="s")

    @functools.partial(
        pl.kernel,
        mesh=mesh,
        out_type=jax.ShapeDtypeStruct((F, B), jnp.float32),
        compiler_params=pltpu.CompilerParams(needs_layout_passes=False),
        scratch_types=[
            pltpu.VMEM((_LANES,), jnp.float32),
            pltpu.VMEM((_LANES,), jnp.float32),
            pltpu.VMEM((_LANES,), jnp.float32),
            pltpu.VMEM((chunk,), jnp.float32),
            pltpu.VMEM((chunk,), jnp.float32),
            pltpu.VMEM((chunk,), jnp.float32),
            pltpu.VMEM((chunk,), jnp.float32),
            pltpu.SemaphoreType.DMA,
            pltpu.SemaphoreType.DMA,
            pltpu.SemaphoreType.DMA,
            pltpu.SemaphoreType.DMA,
        ],
    )
    def run(x_hbm, c_hbm, out_hbm, th_v, ts_v, tr_v,
            xb0, xb1, ob0, ob1, si0, si1, so0, so1):
        wid = lax.axis_index("s") * 2 + lax.axis_index("c")
        f = wid // w_per_f
        col0 = (wid % w_per_f) * cols_per_w
        pltpu.sync_copy(c_hbm.at[f, pl.ds(0, _LANES)], th_v)
        pltpu.sync_copy(c_hbm.at[f, pl.ds(_LANES, _LANES)], ts_v)
        pltpu.sync_copy(c_hbm.at[f, pl.ds(2 * _LANES, _LANES)], tr_v)
        thv = th_v[...]
        th7, th3, th11 = thv[7], thv[3], thv[11]

        def compute(xbuf, obuf):
            @plsc.parallel_loop(0, chunk // _LANES, 1, unroll=4)
            def vec_body(r):
                xv = xbuf[pl.ds(r * _LANES, _LANES)]
                m0 = xv > th7
                enc = jnp.where(m0, jnp.int32(8), jnp.int32(0))
                pv = jnp.where(m0, th11, th3)
                enc = jnp.where(xv > pv, enc + 4, enc)
                pv = plsc.load_gather(ts_v, [enc])
                enc = jnp.where(xv > pv, enc + 2, enc)
                pv = plsc.load_gather(th_v, [enc])
                enc = jnp.where(xv > pv, enc + 1, enc)
                obuf[pl.ds(r * _LANES, _LANES)] = plsc.load_gather(tr_v, [enc])

        def drain(sem, buf):
            pltpu.make_async_copy(x_hbm.at[f, pl.ds(col0, chunk)], buf,
                                  sem).wait()

        last = col0 + (n_chunks - 1) * chunk
        pltpu.async_copy(x_hbm.at[f, pl.ds(col0, chunk)], xb0, si0)
        pltpu.async_copy(x_hbm.at[f, pl.ds(col0 + chunk, chunk)], xb1, si1)

        def pair_body(g, carry):
            a0 = col0 + (2 * g) * chunk

            drain(si0, xb0)

            @pl.when(g > 0)
            def _():
                drain(so0, ob0)

            compute(xb0, ob0)
            pltpu.async_copy(ob0, out_hbm.at[f, pl.ds(a0, chunk)], so0)
            nxt0 = jnp.minimum(a0 + 2 * chunk, last)
            pltpu.async_copy(x_hbm.at[f, pl.ds(nxt0, chunk)], xb0, si0)

            drain(si1, xb1)

            @pl.when(g > 0)
            def _():
                drain(so1, ob1)

            compute(xb1, ob1)
            pltpu.async_copy(ob1, out_hbm.at[f, pl.ds(a0 + chunk, chunk)],
                             so1)
            nxt1 = jnp.minimum(a0 + 3 * chunk, last)
            pltpu.async_copy(x_hbm.at[f, pl.ds(nxt1, chunk)], xb1, si1)
            return carry

        lax.fori_loop(0, n_chunks // 2, pair_body, 0)
        drain(si0, xb0)
        drain(si1, xb1)
        drain(so0, ob0)
        drain(so1, ob1)

    return run(xt, consts)


def kernel(x, thresholds):
    B, F = x.shape
    consts = _build_consts(thresholds)
    info = plsc.get_sparse_core_info()
    n_workers = info.num_cores * info.num_subcores
    cols_per_w = B // (n_workers // F)
    chunk = 8192
    out_t = _run(x.T, consts, cols_per_w, chunk)
    return out_t.T
